# Initial kernel scaffold; baseline (speedup 1.0000x reference)
import jax
import jax.numpy as jnp
from jax.experimental import pallas as pl


def kernel(table, vol_idx, dim):
    raise NotImplementedError("write your pallas kernel here")


# speedup vs baseline: 1.4889x; 1.4889x over previous
import functools

import jax
import jax.numpy as jnp
from jax import lax
from jax.experimental import pallas as pl
from jax.experimental.pallas import tpu as pltpu
from jax.experimental.pallas import tpu_sc as plsc

NC = 2
NS = 16
L = 16
NW = NC * NS


def _gather_body(table_hbm, vol_hbm, dim_hbm, out_hbm,
                 vol_v, dim_v, buf0, buf1,
                 sem_g0, sem_g1, sem_s0, sem_s1):
    B = out_hbm.shape[0]
    rpw = B // NW
    nch = rpw // L

    wid = lax.axis_index("s") * NC + lax.axis_index("c")
    base = wid * rpw

    pltpu.sync_copy(vol_hbm.at[pl.ds(base, rpw)], vol_v)
    pltpu.sync_copy(dim_hbm, dim_v)
    dimv = dim_v[...]
    iota = lax.broadcasted_iota(jnp.int32, (L,), 0)

    def idx_vec(j):
        pos = iota + (base + j * L)
        v = vol_v[pl.ds(j * L, L)]
        return jnp.where(pos < dimv, v, jnp.zeros_like(v))

    bufs = (buf0, buf1)
    sg = (sem_g0, sem_g1)
    ss = (sem_s0, sem_s1)
    hg = [None, None]
    hs = [None, None]

    h = pltpu.make_async_copy(table_hbm.at[idx_vec(0)], bufs[0], sg[0])
    h.start()
    hg[0] = h
    for i in range(nch):
        b = i % 2
        hg[b].wait()
        if i + 1 < nch:
            nb = (i + 1) % 2
            if i >= 1:
                hs[nb].wait()
            h = pltpu.make_async_copy(table_hbm.at[idx_vec(i + 1)],
                                      bufs[nb], sg[nb])
            h.start()
            hg[nb] = h
        h = pltpu.make_async_copy(bufs[b],
                                  out_hbm.at[pl.ds(base + i * L, L)], ss[b])
        h.start()
        hs[b] = h
    hs[(nch - 2) % 2].wait()
    hs[(nch - 1) % 2].wait()


def kernel(table, vol_idx, dim):
    B = vol_idx.shape[0] - 1
    D = table.shape[1]
    rpw = B // NW
    dim_vec = jnp.full((L,), dim, dtype=jnp.int32)

    gather = pl.kernel(
        _gather_body,
        out_type=jax.ShapeDtypeStruct((B, D), table.dtype),
        mesh=plsc.VectorSubcoreMesh(core_axis_name="c", subcore_axis_name="s"),
        scratch_types=[
            pltpu.VMEM((rpw,), jnp.int32),
            pltpu.VMEM((L,), jnp.int32),
            pltpu.VMEM((L, D), jnp.float32),
            pltpu.VMEM((L, D), jnp.float32),
            pltpu.SemaphoreType.DMA,
            pltpu.SemaphoreType.DMA,
            pltpu.SemaphoreType.DMA,
            pltpu.SemaphoreType.DMA,
        ],
    )
    out = gather(table, vol_idx.astype(jnp.int32), dim_vec)
    return out[None, ...]


# speedup vs baseline: 1.4934x; 1.0030x over previous
import functools

import jax
import jax.numpy as jnp
from jax import lax
from jax.experimental import pallas as pl
from jax.experimental.pallas import tpu as pltpu
from jax.experimental.pallas import tpu_sc as plsc

NC = 2
NS = 16
L = 16
NW = NC * NS


def _gather_body(table_hbm, vol_hbm, dim_hbm, out_hbm,
                 vol_v, dim_v, idx_v, buf0, buf1,
                 sem_g0, sem_g1, sem_s0, sem_s1):
    B = out_hbm.shape[0]
    rpw = B // NW
    nch = rpw // L

    wid = lax.axis_index("s") * NC + lax.axis_index("c")
    base = wid * rpw

    pltpu.sync_copy(vol_hbm.at[pl.ds(base, rpw)], vol_v)
    pltpu.sync_copy(dim_hbm, dim_v)
    dimv = dim_v[...]
    iota = lax.broadcasted_iota(jnp.int32, (L,), 0)

    for j in range(nch):
        pos = iota + (base + j * L)
        v = vol_v[pl.ds(j * L, L)]
        idx_v[pl.ds(j * L, L)] = jnp.where(pos < dimv, v, jnp.zeros_like(v))

    bufs = (buf0, buf1)
    sg = (sem_g0, sem_g1)
    ss = (sem_s0, sem_s1)
    hg = [None, None]
    hs = [None, None]

    def gather(i, b):
        h = pltpu.make_async_copy(table_hbm.at[idx_v.at[pl.ds(i * L, L)]],
                                  bufs[b], sg[b])
        h.start()
        hg[b] = h

    gather(0, 0)
    for i in range(nch):
        b = i % 2
        hg[b].wait()
        if i + 1 < nch:
            nb = (i + 1) % 2
            if i >= 1:
                hs[nb].wait()
            gather(i + 1, nb)
        h = pltpu.make_async_copy(bufs[b],
                                  out_hbm.at[pl.ds(base + i * L, L)], ss[b])
        h.start()
        hs[b] = h
    hs[(nch - 2) % 2].wait()
    hs[(nch - 1) % 2].wait()


def kernel(table, vol_idx, dim):
    B = vol_idx.shape[0] - 1
    D = table.shape[1]
    rpw = B // NW
    dim_vec = jnp.full((L,), dim, dtype=jnp.int32)

    gather = pl.kernel(
        _gather_body,
        out_type=jax.ShapeDtypeStruct((B, D), table.dtype),
        mesh=plsc.VectorSubcoreMesh(core_axis_name="c", subcore_axis_name="s"),
        scratch_types=[
            pltpu.VMEM((rpw,), jnp.int32),
            pltpu.VMEM((L,), jnp.int32),
            pltpu.VMEM((rpw,), jnp.int32),
            pltpu.VMEM((L, D), jnp.float32),
            pltpu.VMEM((L, D), jnp.float32),
            pltpu.SemaphoreType.DMA,
            pltpu.SemaphoreType.DMA,
            pltpu.SemaphoreType.DMA,
            pltpu.SemaphoreType.DMA,
        ],
    )
    out = gather(table, vol_idx.astype(jnp.int32), dim_vec)
    return out[None, ...]
